# baseline (speedup 1.0000x reference)
import functools

import jax
import jax.numpy as jnp
from jax import lax
from jax.experimental import pallas as pl
from jax.experimental.pallas import tpu as pltpu
from jax.experimental.pallas import tpu_sc as plsc

BATCH = 4096
SEQ = 200
DIM = 64
VOCAB = 100000
L = 16

NC, NS = 2, 16
NW = NC * NS
ROWS_PER_W = BATCH // NW

NCHUNK = 13
TAIL_OFF = SEQ - L

CHUNK = 512
NCH = (VOCAB + CHUNK - 1) // CHUNK
VPAD = NCH * CHUNK
NSLOT = ROWS_PER_W * NCHUNK * L
CSZ = (NCH + 1) * L


def _body(idx_hbm, mask_hbm, w_hbm, out_hbm,
          idx_v, mask_v, payload, counts2d, base2d, woff2d,
          chunk_v, out_v, denom_v, sem):
    wid = lax.axis_index("s") * NC + lax.axis_index("c")
    base = wid * ROWS_PER_W

    pltpu.sync_copy(idx_hbm.at[pl.ds(base, ROWS_PER_W)], idx_v)
    pltpu.sync_copy(mask_hbm.at[pl.ds(base, ROWS_PER_W)], mask_v)

    lane = lax.iota(jnp.int32, L)
    tail_keep = (lane >= (L - (SEQ - (NCHUNK - 1) * L))).astype(jnp.int32)
    ones_i = jnp.ones((L,), jnp.int32)
    zero_i = jnp.zeros((L,), jnp.int32)
    zero_f = jnp.zeros((L,), jnp.float32)

    def zero_counts(i, carry):
        counts2d[pl.ds(i * L, L)] = zero_i
        return carry

    lax.fori_loop(0, NCH + 1, zero_counts, 0)

    def zero_out(r, carry):
        for c in range(DIM // L):
            out_v[r, pl.ds(c * L, L)] = zero_f
        return carry

    lax.fori_loop(0, ROWS_PER_W, zero_out, 0)

    def pass1(r, carry):
        rowcnt = zero_i
        for j in range(NCHUNK):
            off = j * L if j < NCHUNK - 1 else TAIL_OFF
            iv = idx_v[r, pl.ds(off, L)]
            mv = mask_v[r, pl.ds(off, L)]
            if j == NCHUNK - 1:
                mv = mv * tail_keep
            pos = ((iv // CHUNK) * L) + lane
            plsc.addupdate_scatter(counts2d, [pos], ones_i, mask=mv > 0)
            rowcnt = rowcnt + mv
        cnt = jnp.sum(rowcnt).astype(jnp.float32)
        cnt_vec = lax.broadcast_in_dim(cnt, (L,), ())
        denom_v[r, pl.ds(0, L)] = jnp.maximum(cnt_vec, 1e-9)
        return carry

    lax.fori_loop(0, ROWS_PER_W, pass1, 0)

    def mk_base(cid, run):
        c16 = counts2d[pl.ds(cid * L, L)]
        inc = plsc.cumsum(c16)
        run_vec = lax.broadcast_in_dim(run, (L,), ())
        b = run_vec + inc - c16
        base2d[pl.ds(cid * L, L)] = b
        woff2d[pl.ds(cid * L, L)] = b
        return run + jnp.sum(c16)

    total = lax.fori_loop(0, NCH, mk_base, jnp.int32(0))
    base2d[pl.ds(NCH * L, L)] = lax.broadcast_in_dim(total, (L,), ())

    def pass2(r, carry):
        for j in range(NCHUNK):
            off = j * L if j < NCHUNK - 1 else TAIL_OFF
            iv = idx_v[r, pl.ds(off, L)]
            mv = mask_v[r, pl.ds(off, L)]
            if j == NCHUNK - 1:
                mv = mv * tail_keep
            cidpos = ((iv // CHUNK) * L) + lane
            pos = plsc.load_gather(woff2d, [cidpos])
            pval = ((iv % CHUNK) * (ROWS_PER_W)) + r
            plsc.store_scatter(payload, [pos], pval, mask=mv > 0)
            plsc.addupdate_scatter(woff2d, [cidpos], ones_i, mask=mv > 0)
        return carry

    lax.fori_loop(0, ROWS_PER_W, pass2, 0)

    def chunk_body(ci, carry):
        pltpu.sync_copy(w_hbm.at[pl.ds(ci * CHUNK, CHUNK)], chunk_v)
        s = base2d[pl.ds(ci * L, L)][0]
        e = base2d[pl.ds((ci + 1) * L, L)][0]

        def entry(k, c2):
            pv = payload[pl.ds(k, L)][0]
            r = pv % ROWS_PER_W
            il = pv // ROWS_PER_W
            for c in range(DIM // L):
                out_v[r, pl.ds(c * L, L)] = (
                    out_v[r, pl.ds(c * L, L)] + chunk_v[il, pl.ds(c * L, L)])
            return c2

        lax.fori_loop(s, e, entry, 0)
        return carry

    lax.fori_loop(0, NCH, chunk_body, 0)

    def fin(r, carry):
        d = denom_v[r, pl.ds(0, L)]
        for c in range(DIM // L):
            out_v[r, pl.ds(c * L, L)] = out_v[r, pl.ds(c * L, L)] / d
        return carry

    lax.fori_loop(0, ROWS_PER_W, fin, 0)
    pltpu.sync_copy(out_v, out_hbm.at[pl.ds(base, ROWS_PER_W)])


@jax.jit
def _run(idx, mask_idx, W):
    w_pad = jnp.pad(W, ((0, VPAD - VOCAB), (0, 0)))
    mesh = plsc.VectorSubcoreMesh(core_axis_name="c", subcore_axis_name="s")
    return pl.kernel(
        _body,
        mesh=mesh,
        out_type=jax.ShapeDtypeStruct((BATCH, DIM), jnp.float32),
        compiler_params=pltpu.CompilerParams(
            use_tc_tiling_on_sc=False,
            needs_layout_passes=False,
        ),
        scratch_types=[
            pltpu.VMEM((ROWS_PER_W, SEQ), jnp.int32),
            pltpu.VMEM((ROWS_PER_W, SEQ), jnp.int32),
            pltpu.VMEM((NSLOT + L,), jnp.int32),
            pltpu.VMEM((CSZ,), jnp.int32),
            pltpu.VMEM((CSZ,), jnp.int32),
            pltpu.VMEM((CSZ,), jnp.int32),
            pltpu.VMEM((CHUNK, DIM), jnp.float32),
            pltpu.VMEM((ROWS_PER_W, DIM), jnp.float32),
            pltpu.VMEM((ROWS_PER_W, L), jnp.float32),
            pltpu.SemaphoreType.DMA,
        ],
    )(idx, mask_idx, w_pad)


def kernel(idx, mask_idx, W):
    return _run(idx, mask_idx, W)


# speedup vs baseline: 1.2527x; 1.2527x over previous
import functools

import jax
import jax.numpy as jnp
from jax import lax
from jax.experimental import pallas as pl
from jax.experimental.pallas import tpu as pltpu
from jax.experimental.pallas import tpu_sc as plsc

BATCH = 4096
SEQ = 200
DIM = 64
VOCAB = 100000
L = 16

NC, NS = 2, 16
NW = NC * NS
ROWS_PER_W = BATCH // NW
HRW = ROWS_PER_W // 2

NCHUNK = 13
TAIL_OFF = SEQ - L

CHUNK = 384
NCH = 262
VPAD = NCH * CHUNK
NSLOT = ROWS_PER_W * NCHUNK * L
CSZ = (NCH + 1) * L


def _body(idx_hbm, mask_hbm, w_hbm, out_hbm,
          idx_v, mask_v, payload, counts2d, base2d, woff2d,
          chunk_a, chunk_b, out_v, denom_v, sem_a, sem_b):
    wid = lax.axis_index("s") * NC + lax.axis_index("c")
    base = wid * ROWS_PER_W

    lane = lax.iota(jnp.int32, L)
    tail_keep = (lane >= (L - (SEQ - (NCHUNK - 1) * L))).astype(jnp.int32)
    ones_i = jnp.ones((L,), jnp.int32)
    zero_i = jnp.zeros((L,), jnp.int32)
    zero_f = jnp.zeros((L,), jnp.float32)

    def zero_counts(i, carry):
        counts2d[pl.ds(i * L, L)] = zero_i
        return carry

    lax.fori_loop(0, NCH + 1, zero_counts, 0)

    def zero_out(r, carry):
        for c in range(DIM // L):
            out_v[r, pl.ds(c * L, L)] = zero_f
        return carry

    lax.fori_loop(0, ROWS_PER_W, zero_out, 0)

    for hb in range(2):
        pltpu.sync_copy(idx_hbm.at[pl.ds(base + hb * HRW, HRW)], idx_v)
        pltpu.sync_copy(mask_hbm.at[pl.ds(base + hb * HRW, HRW)], mask_v)

        def pass1(r, carry, hb=hb):
            rowcnt = zero_i
            for j in range(NCHUNK):
                off = j * L if j < NCHUNK - 1 else TAIL_OFF
                iv = idx_v[r, pl.ds(off, L)]
                mv = mask_v[r, pl.ds(off, L)]
                if j == NCHUNK - 1:
                    mv = mv * tail_keep
                pos = ((iv // CHUNK) * L) + lane
                plsc.addupdate_scatter(counts2d, [pos], ones_i, mask=mv > 0)
                rowcnt = rowcnt + mv
            cnt = jnp.sum(rowcnt).astype(jnp.float32)
            cnt_vec = lax.broadcast_in_dim(cnt, (L,), ())
            denom_v[hb * HRW + r, pl.ds(0, L)] = jnp.maximum(cnt_vec, 1e-9)
            return carry

        lax.fori_loop(0, HRW, pass1, 0)

    def mk_base(cid, run):
        c16 = counts2d[pl.ds(cid * L, L)]
        inc = plsc.cumsum(c16)
        run_vec = lax.broadcast_in_dim(run, (L,), ())
        b = run_vec + inc - c16
        base2d[pl.ds(cid * L, L)] = b
        woff2d[pl.ds(cid * L, L)] = b
        return run + jnp.sum(c16)

    total = lax.fori_loop(0, NCH, mk_base, jnp.int32(0))
    base2d[pl.ds(NCH * L, L)] = lax.broadcast_in_dim(total, (L,), ())

    for hb in range(2):
        pltpu.sync_copy(idx_hbm.at[pl.ds(base + hb * HRW, HRW)], idx_v)
        pltpu.sync_copy(mask_hbm.at[pl.ds(base + hb * HRW, HRW)], mask_v)

        def pass2(r, carry, hb=hb):
            for j in range(NCHUNK):
                off = j * L if j < NCHUNK - 1 else TAIL_OFF
                iv = idx_v[r, pl.ds(off, L)]
                mv = mask_v[r, pl.ds(off, L)]
                if j == NCHUNK - 1:
                    mv = mv * tail_keep
                cidpos = ((iv // CHUNK) * L) + lane
                pos = plsc.load_gather(woff2d, [cidpos])
                pval = ((iv % CHUNK) * ROWS_PER_W) + (hb * HRW + r)
                plsc.store_scatter(payload, [pos], pval, mask=mv > 0)
                plsc.addupdate_scatter(woff2d, [cidpos], ones_i, mask=mv > 0)
            return carry

        lax.fori_loop(0, HRW, pass2, 0)

    def process(ci, chunk_v):
        s = base2d[pl.ds(ci * L, L)][0]
        e = base2d[pl.ds((ci + 1) * L, L)][0]

        def entry(k, c2):
            pv = payload[pl.ds(k, L)][0]
            r = pv % ROWS_PER_W
            il = pv // ROWS_PER_W
            for c in range(DIM // L):
                out_v[r, pl.ds(c * L, L)] = (
                    out_v[r, pl.ds(c * L, L)] + chunk_v[il, pl.ds(c * L, L)])
            return c2

        lax.fori_loop(s, e, entry, 0)

    pltpu.async_copy(w_hbm.at[pl.ds(0, CHUNK)], chunk_a, sem_a)

    def pair(p, carry):
        ci0 = p * 2
        ci1 = ci0 + 1
        pltpu.async_copy(w_hbm.at[pl.ds(ci1 * CHUNK, CHUNK)], chunk_b, sem_b)
        pltpu.make_async_copy(
            w_hbm.at[pl.ds(ci0 * CHUNK, CHUNK)], chunk_a, sem_a).wait()
        process(ci0, chunk_a)
        nxt = jnp.minimum(ci0 + 2, NCH - 1)
        pltpu.async_copy(w_hbm.at[pl.ds(nxt * CHUNK, CHUNK)], chunk_a, sem_a)
        pltpu.make_async_copy(
            w_hbm.at[pl.ds(ci1 * CHUNK, CHUNK)], chunk_b, sem_b).wait()
        process(ci1, chunk_b)
        return carry

    lax.fori_loop(0, NCH // 2, pair, 0)
    pltpu.make_async_copy(
        w_hbm.at[pl.ds((NCH - 1) * CHUNK, CHUNK)], chunk_a, sem_a).wait()

    def fin(r, carry):
        d = denom_v[r, pl.ds(0, L)]
        for c in range(DIM // L):
            out_v[r, pl.ds(c * L, L)] = out_v[r, pl.ds(c * L, L)] / d
        return carry

    lax.fori_loop(0, ROWS_PER_W, fin, 0)
    pltpu.sync_copy(out_v, out_hbm.at[pl.ds(base, ROWS_PER_W)])


@jax.jit
def _run(idx, mask_idx, W):
    w_pad = jnp.pad(W, ((0, VPAD - VOCAB), (0, 0)))
    mesh = plsc.VectorSubcoreMesh(core_axis_name="c", subcore_axis_name="s")
    return pl.kernel(
        _body,
        mesh=mesh,
        out_type=jax.ShapeDtypeStruct((BATCH, DIM), jnp.float32),
        compiler_params=pltpu.CompilerParams(
            use_tc_tiling_on_sc=False,
            needs_layout_passes=False,
        ),
        scratch_types=[
            pltpu.VMEM((HRW, SEQ), jnp.int32),
            pltpu.VMEM((HRW, SEQ), jnp.int32),
            pltpu.VMEM((NSLOT + L,), jnp.int32),
            pltpu.VMEM((CSZ,), jnp.int32),
            pltpu.VMEM((CSZ,), jnp.int32),
            pltpu.VMEM((CSZ,), jnp.int32),
            pltpu.VMEM((CHUNK, DIM), jnp.float32),
            pltpu.VMEM((CHUNK, DIM), jnp.float32),
            pltpu.VMEM((ROWS_PER_W, DIM), jnp.float32),
            pltpu.VMEM((ROWS_PER_W, L), jnp.float32),
            pltpu.SemaphoreType.DMA,
            pltpu.SemaphoreType.DMA,
        ],
    )(idx, mask_idx, w_pad)


def kernel(idx, mask_idx, W):
    return _run(idx, mask_idx, W)


# speedup vs baseline: 1.5847x; 1.2650x over previous
import functools

import jax
import jax.numpy as jnp
from jax import lax
from jax.experimental import pallas as pl
from jax.experimental.pallas import tpu as pltpu
from jax.experimental.pallas import tpu_sc as plsc

BATCH = 4096
SEQ = 200
DIM = 64
VOCAB = 100000
L = 16

NC, NS = 2, 16
NW = NC * NS
ROWS_PER_W = BATCH // NW
HRW = ROWS_PER_W // 2

NCHUNK = 13
TAIL_OFF = SEQ - L

CHUNK = 256
CSH = 8
NCH = 392
VPAD = NCH * CHUNK
NSLOT = ROWS_PER_W * NCHUNK * L
CSZ = (NCH + 1) * L


def _body(idx_hbm, mask_hbm, w_hbm, out_hbm,
          idx_v, mask_v, payload, counts2d, base2d, woff2d,
          chunk_a, chunk_b, out_v, denom_v, sem_a, sem_b):
    wid = lax.axis_index("s") * NC + lax.axis_index("c")
    base = wid * ROWS_PER_W

    lane = lax.iota(jnp.int32, L)
    tail_keep = (lane >= (L - (SEQ - (NCHUNK - 1) * L))).astype(jnp.int32)
    ones_i = jnp.ones((L,), jnp.int32)
    zero_i = jnp.zeros((L,), jnp.int32)
    zero_f = jnp.zeros((L,), jnp.float32)

    def zero_counts(i, carry):
        counts2d[pl.ds(i * L, L)] = zero_i
        return carry

    lax.fori_loop(0, NCH + 1, zero_counts, 0)

    def zero_out(r, carry):
        for c in range(DIM // L):
            out_v[r, pl.ds(c * L, L)] = zero_f
        return carry

    lax.fori_loop(0, ROWS_PER_W, zero_out, 0)

    for hb in range(2):
        pltpu.sync_copy(idx_hbm.at[pl.ds(base + hb * HRW, HRW)], idx_v)
        pltpu.sync_copy(mask_hbm.at[pl.ds(base + hb * HRW, HRW)], mask_v)

        def pass1(r, carry, hb=hb):
            rowcnt = zero_i
            for j in range(NCHUNK):
                off = j * L if j < NCHUNK - 1 else TAIL_OFF
                iv = idx_v[r, pl.ds(off, L)]
                mv = mask_v[r, pl.ds(off, L)]
                if j == NCHUNK - 1:
                    mv = mv * tail_keep
                pos = (lax.shift_right_logical(iv, CSH) * L) + lane
                plsc.addupdate_scatter(counts2d, [pos], ones_i, mask=mv > 0)
                rowcnt = rowcnt + mv
            cnt = jnp.sum(rowcnt).astype(jnp.float32)
            cnt_vec = lax.broadcast_in_dim(cnt, (L,), ())
            denom_v[hb * HRW + r, pl.ds(0, L)] = jnp.maximum(cnt_vec, 1e-9)
            return carry

        lax.fori_loop(0, HRW, pass1, 0)

    def mk_base(cid, run):
        c16 = counts2d[pl.ds(cid * L, L)]
        inc = plsc.cumsum(c16)
        run_vec = lax.broadcast_in_dim(run, (L,), ())
        b = run_vec + inc - c16
        base2d[pl.ds(cid * L, L)] = b
        woff2d[pl.ds(cid * L, L)] = b
        return run + jnp.sum(c16)

    total = lax.fori_loop(0, NCH, mk_base, jnp.int32(0))
    base2d[pl.ds(NCH * L, L)] = lax.broadcast_in_dim(total, (L,), ())

    for hb in range(2):
        pltpu.sync_copy(idx_hbm.at[pl.ds(base + hb * HRW, HRW)], idx_v)
        pltpu.sync_copy(mask_hbm.at[pl.ds(base + hb * HRW, HRW)], mask_v)

        def pass2(r, carry, hb=hb):
            for j in range(NCHUNK):
                off = j * L if j < NCHUNK - 1 else TAIL_OFF
                iv = idx_v[r, pl.ds(off, L)]
                mv = mask_v[r, pl.ds(off, L)]
                if j == NCHUNK - 1:
                    mv = mv * tail_keep
                cidpos = (lax.shift_right_logical(iv, CSH) * L) + lane
                pos = plsc.load_gather(woff2d, [cidpos])
                pval = (jnp.bitwise_and(iv, CHUNK - 1) * ROWS_PER_W) + (
                    hb * HRW + r)
                plsc.store_scatter(payload, [pos], pval, mask=mv > 0)
                plsc.addupdate_scatter(woff2d, [cidpos], ones_i, mask=mv > 0)
            return carry

        lax.fori_loop(0, HRW, pass2, 0)

    def process(ci, chunk_v):
        s = base2d[pl.ds(ci * L, L)][0]
        e = base2d[pl.ds((ci + 1) * L, L)][0]

        def entry(k, c2):
            pv = payload[pl.ds(k, L)][0]
            r = jnp.bitwise_and(pv, ROWS_PER_W - 1)
            il = lax.shift_right_logical(pv, 7)
            for c in range(DIM // L):
                plsc.addupdate(out_v.at[r, pl.ds(c * L, L)],
                               chunk_v[il, pl.ds(c * L, L)])
            return c2

        lax.fori_loop(s, e, entry, 0)

    pltpu.async_copy(w_hbm.at[pl.ds(0, CHUNK)], chunk_a, sem_a)

    def pair(p, carry):
        ci0 = p * 2
        ci1 = ci0 + 1
        pltpu.async_copy(w_hbm.at[pl.ds(ci1 * CHUNK, CHUNK)], chunk_b, sem_b)
        pltpu.make_async_copy(
            w_hbm.at[pl.ds(ci0 * CHUNK, CHUNK)], chunk_a, sem_a).wait()
        process(ci0, chunk_a)
        nxt = jnp.minimum(ci0 + 2, NCH - 1)
        pltpu.async_copy(w_hbm.at[pl.ds(nxt * CHUNK, CHUNK)], chunk_a, sem_a)
        pltpu.make_async_copy(
            w_hbm.at[pl.ds(ci1 * CHUNK, CHUNK)], chunk_b, sem_b).wait()
        process(ci1, chunk_b)
        return carry

    lax.fori_loop(0, NCH // 2, pair, 0)
    pltpu.make_async_copy(
        w_hbm.at[pl.ds((NCH - 1) * CHUNK, CHUNK)], chunk_a, sem_a).wait()

    def fin(r, carry):
        d = denom_v[r, pl.ds(0, L)]
        for c in range(DIM // L):
            out_v[r, pl.ds(c * L, L)] = out_v[r, pl.ds(c * L, L)] / d
        return carry

    lax.fori_loop(0, ROWS_PER_W, fin, 0)
    pltpu.sync_copy(out_v, out_hbm.at[pl.ds(base, ROWS_PER_W)])


@jax.jit
def _run(idx, mask_idx, W):
    w_pad = jnp.pad(W, ((0, VPAD - VOCAB), (0, 0)))
    mesh = plsc.VectorSubcoreMesh(core_axis_name="c", subcore_axis_name="s")
    return pl.kernel(
        _body,
        mesh=mesh,
        out_type=jax.ShapeDtypeStruct((BATCH, DIM), jnp.float32),
        compiler_params=pltpu.CompilerParams(
            use_tc_tiling_on_sc=False,
            needs_layout_passes=False,
        ),
        scratch_types=[
            pltpu.VMEM((HRW, SEQ), jnp.int32),
            pltpu.VMEM((HRW, SEQ), jnp.int32),
            pltpu.VMEM((NSLOT + L,), jnp.int32),
            pltpu.VMEM((CSZ,), jnp.int32),
            pltpu.VMEM((CSZ,), jnp.int32),
            pltpu.VMEM((CSZ,), jnp.int32),
            pltpu.VMEM((CHUNK, DIM), jnp.float32),
            pltpu.VMEM((CHUNK, DIM), jnp.float32),
            pltpu.VMEM((ROWS_PER_W, DIM), jnp.float32),
            pltpu.VMEM((ROWS_PER_W, L), jnp.float32),
            pltpu.SemaphoreType.DMA,
            pltpu.SemaphoreType.DMA,
        ],
    )(idx, mask_idx, w_pad)


def kernel(idx, mask_idx, W):
    return _run(idx, mask_idx, W)
